# Initial kernel scaffold; baseline (speedup 1.0000x reference)
import jax
import jax.numpy as jnp
from jax.experimental import pallas as pl


def kernel(x, W_word, b_word, pos_table, ln_gamma, ln_beta):
    raise NotImplementedError("write your pallas kernel here")


# speedup vs baseline: 3.1030x; 3.1030x over previous
import functools

import jax
import jax.numpy as jnp
from jax.experimental import pallas as pl
from jax.experimental.pallas import tpu as pltpu

_N_EMBED = 1024
_SEQ = 2048
_BATCH = 4
_BLK = 256


def _body(x_ref, w_ref, b_ref, pos_ref, g_ref, bt_ref, o_ref):
    w = w_ref[0, :]
    c = pos_ref[...] + b_ref[0, :]
    g = g_ref[0, :]
    bt = bt_ref[0, :]
    for b in range(_BATCH):
        xs = x_ref[b, :][:, None]
        v = xs * w + c
        mean = jnp.mean(v, axis=1, keepdims=True)
        d = v - mean
        var = jnp.mean(d * d, axis=1, keepdims=True)
        o_ref[b] = d * jax.lax.rsqrt(var + 1e-12) * g + bt


@jax.jit
def kernel(x, W_word, b_word, pos_table, ln_gamma, ln_beta):
    w2 = W_word.reshape(1, _N_EMBED)
    b2 = b_word.reshape(1, _N_EMBED)
    g2 = ln_gamma.reshape(1, _N_EMBED)
    bt2 = ln_beta.reshape(1, _N_EMBED)
    grid = (_SEQ // _BLK,)
    return pl.pallas_call(
        _body,
        grid=grid,
        in_specs=[
            pl.BlockSpec((_BATCH, _BLK), lambda i: (0, i)),
            pl.BlockSpec((1, _N_EMBED), lambda i: (0, 0)),
            pl.BlockSpec((1, _N_EMBED), lambda i: (0, 0)),
            pl.BlockSpec((_BLK, _N_EMBED), lambda i: (i, 0)),
            pl.BlockSpec((1, _N_EMBED), lambda i: (0, 0)),
            pl.BlockSpec((1, _N_EMBED), lambda i: (0, 0)),
        ],
        out_specs=pl.BlockSpec((_BATCH, _BLK, _N_EMBED), lambda i: (0, i, 0)),
        out_shape=jax.ShapeDtypeStruct((_BATCH, _SEQ, _N_EMBED), jnp.float32),
    )(x, w2, b2, pos_table, g2, bt2)
